# baseline (speedup 1.0000x reference)
import functools

import jax
import jax.numpy as jnp
from jax import lax
from jax.experimental import pallas as pl
from jax.experimental.pallas import tpu as pltpu
from jax.experimental.pallas import tpu_sc as plsc

BATCH = 16384
DIM = 32
NUM_WORKERS = 32
B_PER_W = BATCH // NUM_WORKERS
CHUNK = 128
NCHUNK = B_PER_W // CHUNK
BLOCKS_PER_CHUNK = CHUNK // 16

S_SCI = 1 << 15
S_PAP = 1 << 18
SH_SCI = 15
SH_PAP = 18


def _pack_body(in0, in1, in2, in3, out_ref):
    x = jnp.concatenate(
        [in0[...], in1[...], in2[...], in3[...]], axis=0)
    out_ref[...] = lax.dot_general(
        x, jnp.eye(128, dtype=jnp.float32), (((0,), (0,)), ((), ())),
        precision=lax.Precision.HIGHEST,
        preferred_element_type=jnp.float32)


def _tc_pack(table_t, s, blkr):
    n = table_t.shape[1]
    k_steps = s // blkr
    max_blk = (n + blkr - 1) // blkr - 1

    def in_map(c):
        return lambda k: (0, jnp.minimum(c * k_steps + k, max_blk))

    return pl.pallas_call(
        _pack_body,
        grid=(k_steps,),
        in_specs=[pl.BlockSpec((32, blkr), in_map(c)) for c in range(4)],
        out_specs=pl.BlockSpec((blkr, 128), lambda k: (k, 0)),
        out_shape=jax.ShapeDtypeStruct((s, 128), jnp.float32),
    )(table_t, table_t, table_t, table_t)


def _gather_body(sid_hbm, pid_hbm, s_packed, p_packed, out_hbm,
                 sid_v, pid_v, srow_v, prow_v, srows, prows, out_v,
                 sem_s, sem_p):
    num_cores = 2
    wid = lax.axis_index("s") * num_cores + lax.axis_index("c")
    base = wid * B_PER_W

    pltpu.sync_copy(sid_hbm.at[pl.ds(base, B_PER_W)], sid_v)
    pltpu.sync_copy(pid_hbm.at[pl.ds(base, B_PER_W)], pid_v)

    for i in range(B_PER_W // 16):
        sv = sid_v[pl.ds(i * 16, 16)]
        pv = pid_v[pl.ds(i * 16, 16)]
        c = i // BLOCKS_PER_CHUNK
        r = i % BLOCKS_PER_CHUNK
        srow_v[c, pl.ds(r * 16, 16)] = jnp.bitwise_and(sv, S_SCI - 1)
        prow_v[c, pl.ds(r * 16, 16)] = jnp.bitwise_and(pv, S_PAP - 1)

    lanes = lax.iota(jnp.int32, 16)

    for c in range(NCHUNK):
        cp_s = pltpu.async_copy(s_packed.at[srow_v.at[c]], srows, sem_s)
        cp_p = pltpu.async_copy(p_packed.at[prow_v.at[c]], prows, sem_p)
        cp_s.wait()
        cp_p.wait()

        def blk_body(lb, _):
            b = c * BLOCKS_PER_CHUNK + lb
            row_idx = lb * 16 + lanes
            sv = sid_v[pl.ds(b * 16, 16)]
            pv = pid_v[pl.ds(b * 16, 16)]
            scol0 = lax.shift_left(lax.shift_right_logical(sv, SH_SCI), 5)
            pcol0 = lax.shift_left(lax.shift_right_logical(pv, SH_PAP), 5)
            acc = jnp.zeros((16,), jnp.float32)
            for d in range(DIM):
                se = plsc.load_gather(srows, [row_idx, scol0 + d])
                pe = plsc.load_gather(prows, [row_idx, pcol0 + d])
                acc = acc + se * pe
            out_v[pl.ds(b * 16, 16)] = acc
            return ()

        lax.fori_loop(0, BLOCKS_PER_CHUNK, blk_body, ())

    pltpu.sync_copy(out_v, out_hbm.at[pl.ds(base, B_PER_W)])


@jax.jit
def kernel(sid, pid, scientist_table, paper_table):
    s_packed = _tc_pack(scientist_table.T, S_SCI, 2048)
    p_packed = _tc_pack(paper_table.T, S_PAP, 8192)
    mesh = plsc.VectorSubcoreMesh(core_axis_name="c", subcore_axis_name="s")
    run = pl.kernel(
        _gather_body,
        out_type=jax.ShapeDtypeStruct((BATCH,), jnp.float32),
        mesh=mesh,
        scratch_types=[
            pltpu.VMEM((B_PER_W,), jnp.int32),
            pltpu.VMEM((B_PER_W,), jnp.int32),
            pltpu.VMEM((NCHUNK, CHUNK), jnp.int32),
            pltpu.VMEM((NCHUNK, CHUNK), jnp.int32),
            pltpu.VMEM((CHUNK, 128), jnp.float32),
            pltpu.VMEM((CHUNK, 128), jnp.float32),
            pltpu.VMEM((B_PER_W,), jnp.float32),
            pltpu.SemaphoreType.DMA,
            pltpu.SemaphoreType.DMA,
        ],
        compiler_params=pltpu.CompilerParams(
            needs_layout_passes=False, use_tc_tiling_on_sc=True),
    )
    return run(sid.astype(jnp.int32), pid.astype(jnp.int32),
               s_packed, p_packed)


# speedup vs baseline: 1.1999x; 1.1999x over previous
import functools

import jax
import jax.numpy as jnp
from jax import lax
from jax.experimental import pallas as pl
from jax.experimental.pallas import tpu as pltpu
from jax.experimental.pallas import tpu_sc as plsc

BATCH = 16384
DIM = 32
NUM_WORKERS = 32
B_PER_W = BATCH // NUM_WORKERS
CHUNK = 128
NCHUNK = B_PER_W // CHUNK
BLOCKS_PER_CHUNK = CHUNK // 16

S_SCI = 1 << 15
S_PAP = 1 << 18
SH_SCI = 15
SH_PAP = 18


def _pack_body(in0, in1, in2, in3, out_ref):
    x = jnp.concatenate(
        [in0[...], in1[...], in2[...], in3[...]], axis=0)
    hi = x.astype(jnp.bfloat16).astype(jnp.float32)
    lo = x - hi
    eye = jnp.eye(128, dtype=jnp.float32)
    dims = (((0,), (0,)), ((), ()))
    out_ref[...] = (
        lax.dot_general(hi, eye, dims, preferred_element_type=jnp.float32)
        + lax.dot_general(lo, eye, dims, preferred_element_type=jnp.float32))


def _tc_pack(table_t, s, blkr):
    n = table_t.shape[1]
    k_steps = s // blkr
    max_blk = (n + blkr - 1) // blkr - 1

    def in_map(c):
        return lambda k: (0, jnp.minimum(c * k_steps + k, max_blk))

    return pl.pallas_call(
        _pack_body,
        grid=(k_steps,),
        in_specs=[pl.BlockSpec((32, blkr), in_map(c)) for c in range(4)],
        out_specs=pl.BlockSpec((blkr, 128), lambda k: (k, 0)),
        out_shape=jax.ShapeDtypeStruct((s, 128), jnp.float32),
    )(table_t, table_t, table_t, table_t)


def _gather_body(sid_hbm, pid_hbm, s_packed, p_packed, out_hbm,
                 sid_v, pid_v, srow_v, prow_v, srows, prows, out_v,
                 sem_s, sem_p):
    num_cores = 2
    wid = lax.axis_index("s") * num_cores + lax.axis_index("c")
    base = wid * B_PER_W

    pltpu.sync_copy(sid_hbm.at[pl.ds(base, B_PER_W)], sid_v)
    pltpu.sync_copy(pid_hbm.at[pl.ds(base, B_PER_W)], pid_v)

    for i in range(B_PER_W // 16):
        sv = sid_v[pl.ds(i * 16, 16)]
        pv = pid_v[pl.ds(i * 16, 16)]
        c = i // BLOCKS_PER_CHUNK
        r = i % BLOCKS_PER_CHUNK
        srow_v[c, pl.ds(r * 16, 16)] = jnp.bitwise_and(sv, S_SCI - 1)
        prow_v[c, pl.ds(r * 16, 16)] = jnp.bitwise_and(pv, S_PAP - 1)

    lanes = lax.iota(jnp.int32, 16)

    for c in range(NCHUNK):
        cp_s = pltpu.async_copy(s_packed.at[srow_v.at[c]], srows, sem_s)
        cp_p = pltpu.async_copy(p_packed.at[prow_v.at[c]], prows, sem_p)
        cp_s.wait()
        cp_p.wait()

        def blk_body(lb, _):
            b = c * BLOCKS_PER_CHUNK + lb
            row_idx = lb * 16 + lanes
            sv = sid_v[pl.ds(b * 16, 16)]
            pv = pid_v[pl.ds(b * 16, 16)]
            scol0 = lax.shift_left(lax.shift_right_logical(sv, SH_SCI), 5)
            pcol0 = lax.shift_left(lax.shift_right_logical(pv, SH_PAP), 5)
            acc = jnp.zeros((16,), jnp.float32)
            for d in range(DIM):
                se = plsc.load_gather(srows, [row_idx, scol0 + d])
                pe = plsc.load_gather(prows, [row_idx, pcol0 + d])
                acc = acc + se * pe
            out_v[pl.ds(b * 16, 16)] = acc
            return ()

        lax.fori_loop(0, BLOCKS_PER_CHUNK, blk_body, ())

    pltpu.sync_copy(out_v, out_hbm.at[pl.ds(base, B_PER_W)])


@jax.jit
def kernel(sid, pid, scientist_table, paper_table):
    s_packed = _tc_pack(scientist_table.T, S_SCI, 2048)
    p_packed = _tc_pack(paper_table.T, S_PAP, 8192)
    mesh = plsc.VectorSubcoreMesh(core_axis_name="c", subcore_axis_name="s")
    run = pl.kernel(
        _gather_body,
        out_type=jax.ShapeDtypeStruct((BATCH,), jnp.float32),
        mesh=mesh,
        scratch_types=[
            pltpu.VMEM((B_PER_W,), jnp.int32),
            pltpu.VMEM((B_PER_W,), jnp.int32),
            pltpu.VMEM((NCHUNK, CHUNK), jnp.int32),
            pltpu.VMEM((NCHUNK, CHUNK), jnp.int32),
            pltpu.VMEM((CHUNK, 128), jnp.float32),
            pltpu.VMEM((CHUNK, 128), jnp.float32),
            pltpu.VMEM((B_PER_W,), jnp.float32),
            pltpu.SemaphoreType.DMA,
            pltpu.SemaphoreType.DMA,
        ],
        compiler_params=pltpu.CompilerParams(
            needs_layout_passes=False, use_tc_tiling_on_sc=True),
    )
    return run(sid.astype(jnp.int32), pid.astype(jnp.int32),
               s_packed, p_packed)


# speedup vs baseline: 1.3012x; 1.0844x over previous
import functools

import jax
import jax.numpy as jnp
from jax import lax
from jax.experimental import pallas as pl
from jax.experimental.pallas import tpu as pltpu
from jax.experimental.pallas import tpu_sc as plsc

BATCH = 16384
DIM = 32
NUM_WORKERS = 32
B_PER_W = BATCH // NUM_WORKERS
CHUNK = 128
NCHUNK = B_PER_W // CHUNK
BLOCKS_PER_CHUNK = CHUNK // 16

S_SCI = 1 << 15
S_PAP = 1 << 18
SH_SCI = 15
SH_PAP = 18


def _pack_body(in0, in1, in2, in3, out_ref):
    x = jnp.concatenate(
        [in0[...], in1[...], in2[...], in3[...]], axis=0)
    hi = x.astype(jnp.bfloat16).astype(jnp.float32)
    lo = x - hi
    eye = jnp.eye(128, dtype=jnp.float32)
    dims = (((0,), (0,)), ((), ()))
    out_ref[...] = (
        lax.dot_general(hi, eye, dims, preferred_element_type=jnp.float32)
        + lax.dot_general(lo, eye, dims, preferred_element_type=jnp.float32))


def _tc_pack(table_t, s, blkr):
    n = table_t.shape[1]
    k_steps = s // blkr
    max_blk = (n + blkr - 1) // blkr - 1

    def in_map(c):
        return lambda k: (0, jnp.minimum(c * k_steps + k, max_blk))

    return pl.pallas_call(
        _pack_body,
        grid=(k_steps,),
        in_specs=[pl.BlockSpec((32, blkr), in_map(c)) for c in range(4)],
        out_specs=pl.BlockSpec((blkr, 128), lambda k: (k, 0)),
        out_shape=jax.ShapeDtypeStruct((s, 128), jnp.float32),
    )(table_t, table_t, table_t, table_t)


def _gather_body(sid_hbm, pid_hbm, s_packed, p_packed, out_hbm,
                 sid_v, pid_v, srow_v, prow_v, srows, prows, out_v,
                 sem_s, sem_p):
    num_cores = 2
    wid = lax.axis_index("s") * num_cores + lax.axis_index("c")
    base = wid * B_PER_W

    pltpu.sync_copy(sid_hbm.at[pl.ds(base, B_PER_W)], sid_v)
    pltpu.sync_copy(pid_hbm.at[pl.ds(base, B_PER_W)], pid_v)

    for i in range(B_PER_W // 16):
        sv = sid_v[pl.ds(i * 16, 16)]
        pv = pid_v[pl.ds(i * 16, 16)]
        c = i // BLOCKS_PER_CHUNK
        r = i % BLOCKS_PER_CHUNK
        srow_v[c, pl.ds(r * 16, 16)] = jnp.bitwise_and(sv, S_SCI - 1)
        prow_v[c, pl.ds(r * 16, 16)] = jnp.bitwise_and(pv, S_PAP - 1)

    lanes = lax.iota(jnp.int32, 16)

    def fire(c, buf):
        cp_s = pltpu.async_copy(s_packed.at[srow_v.at[c]], srows.at[buf],
                                sem_s)
        cp_p = pltpu.async_copy(p_packed.at[prow_v.at[c]], prows.at[buf],
                                sem_p)
        return cp_s, cp_p

    inflight = fire(0, 0)
    for c in range(NCHUNK):
        cp_s, cp_p = inflight
        cp_s.wait()
        cp_p.wait()
        buf = c % 2
        if c + 1 < NCHUNK:
            inflight = fire(c + 1, (c + 1) % 2)

        def blk_body(lb, _):
            b = c * BLOCKS_PER_CHUNK + lb
            row_idx = lb * 16 + lanes
            sv = sid_v[pl.ds(b * 16, 16)]
            pv = pid_v[pl.ds(b * 16, 16)]
            scol0 = lax.shift_left(lax.shift_right_logical(sv, SH_SCI), 5)
            pcol0 = lax.shift_left(lax.shift_right_logical(pv, SH_PAP), 5)
            acc = jnp.zeros((16,), jnp.float32)
            for d in range(DIM):
                se = plsc.load_gather(srows.at[buf], [row_idx, scol0 + d])
                pe = plsc.load_gather(prows.at[buf], [row_idx, pcol0 + d])
                acc = acc + se * pe
            out_v[pl.ds(b * 16, 16)] = acc
            return ()

        lax.fori_loop(0, BLOCKS_PER_CHUNK, blk_body, ())

    pltpu.sync_copy(out_v, out_hbm.at[pl.ds(base, B_PER_W)])


@jax.jit
def kernel(sid, pid, scientist_table, paper_table):
    s_packed = _tc_pack(scientist_table.T, S_SCI, 8192)
    p_packed = _tc_pack(paper_table.T, S_PAP, 8192)
    mesh = plsc.VectorSubcoreMesh(core_axis_name="c", subcore_axis_name="s")
    run = pl.kernel(
        _gather_body,
        out_type=jax.ShapeDtypeStruct((BATCH,), jnp.float32),
        mesh=mesh,
        scratch_types=[
            pltpu.VMEM((B_PER_W,), jnp.int32),
            pltpu.VMEM((B_PER_W,), jnp.int32),
            pltpu.VMEM((NCHUNK, CHUNK), jnp.int32),
            pltpu.VMEM((NCHUNK, CHUNK), jnp.int32),
            pltpu.VMEM((2, CHUNK, 128), jnp.float32),
            pltpu.VMEM((2, CHUNK, 128), jnp.float32),
            pltpu.VMEM((B_PER_W,), jnp.float32),
            pltpu.SemaphoreType.DMA,
            pltpu.SemaphoreType.DMA,
        ],
        compiler_params=pltpu.CompilerParams(
            needs_layout_passes=False, use_tc_tiling_on_sc=True),
    )
    return run(sid.astype(jnp.int32), pid.astype(jnp.int32),
               s_packed, p_packed)
